# baseline (speedup 1.0000x reference)
import functools
import math

import jax
import jax.numpy as jnp
from jax import lax
from jax.experimental import pallas as pl
from jax.experimental.pallas import tpu as pltpu
from jax.experimental.pallas import tpu_sc as plsc

_NC = 2
_NS = 16

_W1 = float(1.0 / (1.0 + math.exp(-0.5)))


def _sc_segment_sum(feat_path, ids, n_nodes):
    E, D = feat_path.shape
    NW = _NC * _NS
    NL = 16
    NV = D // NL
    rt = E // NW
    C = 80
    n_chunks = rt // C
    assert rt % C == 0 and C % 8 == 0 and n_chunks % 2 == 1 and n_chunks >= 7
    FB = C // NL
    zr = (n_nodes // _NS) // 8 * 8
    zl = n_nodes - zr * (_NS - 1)
    dummy = n_nodes

    mesh = plsc.VectorSubcoreMesh(core_axis_name="c", subcore_axis_name="s")

    @functools.partial(
        pl.kernel,
        mesh=mesh,
        out_type=jax.ShapeDtypeStruct((_NC, n_nodes, D), jnp.float32),
        compiler_params=pltpu.CompilerParams(needs_layout_passes=False),
        scratch_types=[
            pltpu.VMEM((C * D,), jnp.float32),
            pltpu.VMEM((C * D,), jnp.float32),
            pltpu.VMEM((C,), jnp.int32),
            pltpu.VMEM((C,), jnp.int32),
            pltpu.VMEM((2, C, D), jnp.float32),
            pltpu.VMEM((C,), jnp.int32),
            pltpu.VMEM((C,), jnp.int32),
            pltpu.VMEM_SHARED((n_nodes + 8, D), jnp.float32),
            [pltpu.SemaphoreType.DMA] * 2,
            [pltpu.SemaphoreType.DMA] * 2,
        ],
    )
    def seg_sum(rows_hbm, ids_hbm, zeros_hbm, out_hbm, rows0, rows1,
                ids0, ids1, stage_r, stage_i0, stage_i1, acc, sem_in, sem_sc):
        cid = lax.axis_index("c")
        sid = lax.axis_index("s")
        wid = cid * _NS + sid
        base = wid * rt
        dummy_vec = jnp.full((NL,), dummy, jnp.int32)
        rows_bufs = (rows0, rows1)
        ids_bufs = (ids0, ids1)
        stage_i_bufs = (stage_i0, stage_i1)

        def start_in(g, b):
            pltpu.async_copy(
                rows_hbm.at[pl.ds((base + g * C) * D, C * D)], rows_bufs[b],
                sem_in[b])
            pltpu.async_copy(
                ids_hbm.at[pl.ds(base + g * C, C)], ids_bufs[b], sem_in[b])

        def wait_in(b):
            pltpu.make_async_copy(
                rows_hbm.at[pl.ds(0, C * D)], rows_bufs[b], sem_in[b]).wait()
            pltpu.make_async_copy(
                ids_hbm.at[pl.ds(0, C)], ids_bufs[b], sem_in[b]).wait()

        def scatter_stage(s, cnt):
            si = stage_i_bufs[s]

            @pl.when(cnt <= NL)
            def _():
                pltpu.async_copy(stage_r.at[s, pl.ds(0, NL)],
                                 acc.at[si[pl.ds(0, NL)]], sem_sc[s], add=True)

            @pl.when(cnt > NL)
            def _():
                for j in range(FB):
                    pltpu.async_copy(stage_r.at[s, pl.ds(j * NL, NL)],
                                     acc.at[si[pl.ds(j * NL, NL)]],
                                     sem_sc[s], add=True)

        def wait_stage(s, cntp):
            @pl.when(cntp <= NL)
            def _():
                pltpu.make_async_copy(stage_r.at[s, pl.ds(0, NL)],
                                      acc.at[dummy_vec], sem_sc[s]).wait()

            @pl.when(cntp > NL)
            def _():
                for j in range(FB):
                    pltpu.make_async_copy(stage_r.at[s, pl.ds(j * NL, NL)],
                                          acc.at[dummy_vec], sem_sc[s]).wait()

        lane0 = lax.iota(jnp.int32, NL) == 0
        iotas = [lax.iota(jnp.int32, NL) + j * NL for j in range(NV)]

        def process_chunk(b, cur, cnt0, accs):
            rows_b = rows_bufs[b]
            ids_b = ids_bufs[b]

            def group(rg, carry):
                cur, cnt = carry[0], carry[1]
                accs = list(carry[2:])
                idvec = ids_b[pl.ds(rg * NL, NL)]
                for u in range(NL):
                    r = rg * NL + u
                    idr = idvec[u]
                    same = idr == cur
                    row = [rows_b[pl.ds(r * D + j * NL, NL)] for j in range(NV)]
                    old = accs

                    @pl.when(jnp.logical_not(same))
                    def _(cur=cur, cnt=cnt, old=old):
                        cnt_vec = jnp.broadcast_to(cnt, (NL,))
                        for j in range(NV):
                            plsc.store_scatter(
                                stage_r.at[b], [cnt_vec, iotas[j]], old[j])
                        plsc.store_scatter(
                            stage_i_bufs[b], [cnt_vec],
                            jnp.broadcast_to(cur, (NL,)), mask=lane0)

                    accs = [jnp.where(same, old[j] + row[j], row[j])
                            for j in range(NV)]
                    cnt = jnp.where(same, cnt, cnt + 1)
                    cur = idr
                return (cur, cnt, *accs)

            out = lax.fori_loop(0, C // NL, group, (cur, cnt0, *accs))
            return out[0], out[1], list(out[2:])

        def prefill(s):
            for j in range(FB):
                plsc.store_scatter(stage_i_bufs[s], [iotas[j]], dummy_vec)

        start_in(0, 0)
        start_in(1, 1)

        @pl.when(sid < _NS - 1)
        def _():
            pltpu.sync_copy(zeros_hbm.at[pl.ds(0, zr)], acc.at[pl.ds(sid * zr, zr)])

        @pl.when(sid == _NS - 1)
        def _():
            pltpu.sync_copy(zeros_hbm.at[pl.ds(0, zl)],
                            acc.at[pl.ds((_NS - 1) * zr, zl)])

        plsc.subcore_barrier()

        wait_in(0)
        cur = ids0[pl.ds(0, NL)][0]
        accs = [jnp.zeros((NL,), jnp.float32)] * NV
        prefill(0)
        cur, cntA, accs = process_chunk(0, cur, jnp.int32(0), accs)
        scatter_stage(0, cntA)
        start_in(2, 0)

        wait_in(1)
        prefill(1)
        cur, cntB, accs = process_chunk(1, cur, jnp.int32(0), accs)
        scatter_stage(1, cntB)
        start_in(3, 1)

        def body(k, carry):
            g = 2 * k
            cur, cntA, cntB = carry[0], carry[1], carry[2]
            accs = list(carry[3:])
            wait_in(0)
            wait_stage(0, cntA)
            prefill(0)
            cur, cntA, accs = process_chunk(0, cur, jnp.int32(0), accs)
            scatter_stage(0, cntA)
            start_in(g + 2, 0)
            wait_in(1)
            wait_stage(1, cntB)
            prefill(1)
            cur, cntB, accs = process_chunk(1, cur, jnp.int32(0), accs)
            scatter_stage(1, cntB)
            start_in(g + 3, 1)
            return (cur, cntA, cntB, *accs)

        out = lax.fori_loop(1, (n_chunks - 3) // 2, body,
                            (cur, cntA, cntB, *accs))
        cur, cntA, cntB = out[0], out[1], out[2]
        accs = list(out[3:])

        wait_in(0)
        wait_stage(0, cntA)
        prefill(0)
        cur, cntA, accs = process_chunk(0, cur, jnp.int32(0), accs)
        scatter_stage(0, cntA)
        start_in(n_chunks - 1, 0)

        wait_in(1)
        wait_stage(1, cntB)
        prefill(1)
        cur, cntB, accs = process_chunk(1, cur, jnp.int32(0), accs)
        scatter_stage(1, cntB)

        wait_in(0)
        wait_stage(0, cntA)
        prefill(0)
        cur, cntA, accs = process_chunk(0, cur, jnp.int32(0), accs)
        scatter_stage(0, cntA)

        wait_stage(1, cntB)
        prefill(1)
        zero_vec = jnp.zeros((NL,), jnp.int32)
        for j in range(NV):
            plsc.store_scatter(stage_r.at[1], [zero_vec, iotas[j]], accs[j])
        plsc.store_scatter(stage_i1, [zero_vec],
                           jnp.broadcast_to(cur, (NL,)), mask=lane0)
        pltpu.async_copy(stage_r.at[1, pl.ds(0, NL)],
                         acc.at[stage_i1[pl.ds(0, NL)]], sem_sc[1], add=True)
        pltpu.make_async_copy(stage_r.at[1, pl.ds(0, NL)],
                              acc.at[dummy_vec], sem_sc[1]).wait()
        wait_stage(0, cntA)

        plsc.subcore_barrier()

        @pl.when(sid < _NS - 1)
        def _():
            pltpu.sync_copy(
                acc.at[pl.ds(sid * zr, zr)], out_hbm.at[cid, pl.ds(sid * zr, zr)])

        @pl.when(sid == _NS - 1)
        def _():
            pltpu.sync_copy(
                acc.at[pl.ds((_NS - 1) * zr, zl)],
                out_hbm.at[cid, pl.ds((_NS - 1) * zr, zl)])

    zeros = jnp.zeros((zl, D), jnp.float32)
    return seg_sum(feat_path.reshape(-1), ids, zeros)


def _tc_dense(partials, feat_center, W_type, W_enc, W_dec, W_cls, b_cls):
    N, D = feat_center.shape
    P = W_type.shape[1]
    S = W_cls.shape[1]
    BN = 2000
    assert N % BN == 0

    def body(p_ref, fc_ref, wt_ref, we_ref, wd_ref, wc_ref, bc_ref, out1_ref, out2_ref):
        wcomb = jnp.dot(we_ref[...], wd_ref[...], preferred_element_type=jnp.float32)
        seg = p_ref[0] + p_ref[1]
        inj = jnp.dot(seg, wcomb, preferred_element_type=jnp.float32) * _W1
        center = jnp.dot(fc_ref[...], wt_ref[...], preferred_element_type=jnp.float32)
        logits = jnp.dot(center + inj, wc_ref[...],
                         preferred_element_type=jnp.float32) + bc_ref[...]
        m = jnp.max(logits, axis=1, keepdims=True)
        lse = jnp.log(jnp.sum(jnp.exp(logits - m), axis=1, keepdims=True)) + m
        out1_ref[...] = logits - lse
        out2_ref[...] = inj

    return pl.pallas_call(
        body,
        grid=(N // BN,),
        in_specs=[
            pl.BlockSpec((2, BN, D), lambda i: (0, i, 0)),
            pl.BlockSpec((BN, D), lambda i: (i, 0)),
            pl.BlockSpec((D, P), lambda i: (0, 0)),
            pl.BlockSpec((D, P), lambda i: (0, 0)),
            pl.BlockSpec((P, P), lambda i: (0, 0)),
            pl.BlockSpec((P, S), lambda i: (0, 0)),
            pl.BlockSpec((1, S), lambda i: (0, 0)),
        ],
        out_specs=[
            pl.BlockSpec((BN, S), lambda i: (i, 0)),
            pl.BlockSpec((BN, P), lambda i: (i, 0)),
        ],
        out_shape=[
            jax.ShapeDtypeStruct((N, S), jnp.float32),
            jax.ShapeDtypeStruct((N, P), jnp.float32),
        ],
    )(partials, feat_center, W_type, W_enc, W_dec, W_cls, b_cls.reshape(1, S))


def kernel(feat_center, feat_path, segment_ids, W_type, W_enc, W_dec, W_cls, b_cls):
    N = feat_center.shape[0]
    ids = segment_ids.astype(jnp.int32)
    partials = _sc_segment_sum(feat_path, ids, N)
    pre_embed, inj_scaled = _tc_dense(
        partials, feat_center, W_type, W_enc, W_dec, W_cls, b_cls)
    return (pre_embed, inj_scaled)


# speedup vs baseline: 1.0027x; 1.0027x over previous
import functools
import math

import jax
import jax.numpy as jnp
from jax import lax
from jax.experimental import pallas as pl
from jax.experimental.pallas import tpu as pltpu
from jax.experimental.pallas import tpu_sc as plsc

_NC = 2
_NS = 16

_W1 = float(1.0 / (1.0 + math.exp(-0.5)))


def _sc_segment_sum(feat_path, ids, n_nodes):
    E, D = feat_path.shape
    NW = _NC * _NS
    NL = 16
    NV = D // NL
    rt = E // NW
    C = 80
    n_chunks = rt // C
    assert rt % C == 0 and C % 8 == 0 and n_chunks % 2 == 1 and n_chunks >= 7
    FB = C // NL
    zr = (n_nodes // _NS) // 8 * 8
    zl = n_nodes - zr * (_NS - 1)
    dummy = n_nodes

    mesh = plsc.VectorSubcoreMesh(core_axis_name="c", subcore_axis_name="s")

    @functools.partial(
        pl.kernel,
        mesh=mesh,
        out_type=jax.ShapeDtypeStruct((_NC, n_nodes, D), jnp.float32),
        compiler_params=pltpu.CompilerParams(needs_layout_passes=False),
        scratch_types=[
            pltpu.VMEM((C * D,), jnp.float32),
            pltpu.VMEM((C * D,), jnp.float32),
            pltpu.VMEM((C,), jnp.int32),
            pltpu.VMEM((C,), jnp.int32),
            pltpu.VMEM((2, C, D), jnp.float32),
            pltpu.VMEM((C,), jnp.int32),
            pltpu.VMEM((C,), jnp.int32),
            pltpu.VMEM_SHARED((n_nodes + 8, D), jnp.float32),
            [pltpu.SemaphoreType.DMA] * 2,
            [pltpu.SemaphoreType.DMA] * 2,
        ],
    )
    def seg_sum(rows_hbm, ids_hbm, zeros_hbm, out_hbm, rows0, rows1,
                ids0, ids1, stage_r, stage_i0, stage_i1, acc, sem_in, sem_sc):
        cid = lax.axis_index("c")
        sid = lax.axis_index("s")
        wid = cid * _NS + sid
        base = wid * rt
        dummy_vec = jnp.full((NL,), dummy, jnp.int32)
        rows_bufs = (rows0, rows1)
        ids_bufs = (ids0, ids1)
        stage_i_bufs = (stage_i0, stage_i1)

        def start_in(g, b):
            pltpu.async_copy(
                rows_hbm.at[pl.ds((base + g * C) * D, C * D)], rows_bufs[b],
                sem_in[b])
            pltpu.async_copy(
                ids_hbm.at[pl.ds(base + g * C, C)], ids_bufs[b], sem_in[b])

        def wait_in(b):
            pltpu.make_async_copy(
                rows_hbm.at[pl.ds(0, C * D)], rows_bufs[b], sem_in[b]).wait()
            pltpu.make_async_copy(
                ids_hbm.at[pl.ds(0, C)], ids_bufs[b], sem_in[b]).wait()

        def scatter_stage(s, cnt):
            si = stage_i_bufs[s]

            @pl.when(cnt <= NL)
            def _():
                pltpu.async_copy(stage_r.at[s, pl.ds(0, NL)],
                                 acc.at[si[pl.ds(0, NL)]], sem_sc[s], add=True)

            @pl.when(cnt > NL)
            def _():
                for j in range(FB):
                    pltpu.async_copy(stage_r.at[s, pl.ds(j * NL, NL)],
                                     acc.at[si[pl.ds(j * NL, NL)]],
                                     sem_sc[s], add=True)

        def wait_stage(s, cntp):
            @pl.when(cntp <= NL)
            def _():
                pltpu.make_async_copy(stage_r.at[s, pl.ds(0, NL)],
                                      acc.at[dummy_vec], sem_sc[s]).wait()

            @pl.when(cntp > NL)
            def _():
                for j in range(FB):
                    pltpu.make_async_copy(stage_r.at[s, pl.ds(j * NL, NL)],
                                          acc.at[dummy_vec], sem_sc[s]).wait()

        lane0 = lax.iota(jnp.int32, NL) == 0
        iotas = [lax.iota(jnp.int32, NL) + j * NL for j in range(NV)]

        def process_chunk(b, cur, cnt0, accs):
            rows_b = rows_bufs[b]
            ids_b = ids_bufs[b]

            def fast(rg, carry):
                cur, cnt = carry[0], carry[1]
                accs = list(carry[2:])
                for u in range(NL):
                    r = rg * NL + u
                    accs = [accs[j] + rows_b[pl.ds(r * D + j * NL, NL)]
                            for j in range(NV)]
                return (cur, cnt, *accs)

            def slow(rg, carry, idvec):
                cur, cnt = carry[0], carry[1]
                accs = list(carry[2:])
                for u in range(NL):
                    r = rg * NL + u
                    idr = idvec[u]
                    same = idr == cur
                    row = [rows_b[pl.ds(r * D + j * NL, NL)] for j in range(NV)]
                    old = accs

                    @pl.when(jnp.logical_not(same))
                    def _(cur=cur, cnt=cnt, old=old):
                        cnt_vec = jnp.broadcast_to(cnt, (NL,))
                        for j in range(NV):
                            plsc.store_scatter(
                                stage_r.at[b], [cnt_vec, iotas[j]], old[j])
                        plsc.store_scatter(
                            stage_i_bufs[b], [cnt_vec],
                            jnp.broadcast_to(cur, (NL,)), mask=lane0)

                    accs = [jnp.where(same, old[j] + row[j], row[j])
                            for j in range(NV)]
                    cnt = jnp.where(same, cnt, cnt + 1)
                    cur = idr
                return (cur, cnt, *accs)

            def group(rg, carry):
                idvec = ids_b[pl.ds(rg * NL, NL)]
                uni = jnp.all(idvec == jnp.broadcast_to(carry[0], (NL,)))
                return lax.cond(uni,
                                lambda ops: fast(rg, ops),
                                lambda ops: slow(rg, ops, idvec),
                                carry)

            out = lax.fori_loop(0, C // NL, group, (cur, cnt0, *accs))
            return out[0], out[1], list(out[2:])

        def prefill(s):
            for j in range(FB):
                plsc.store_scatter(stage_i_bufs[s], [iotas[j]], dummy_vec)

        start_in(0, 0)
        start_in(1, 1)

        @pl.when(sid < _NS - 1)
        def _():
            pltpu.sync_copy(zeros_hbm.at[pl.ds(0, zr)], acc.at[pl.ds(sid * zr, zr)])

        @pl.when(sid == _NS - 1)
        def _():
            pltpu.sync_copy(zeros_hbm.at[pl.ds(0, zl)],
                            acc.at[pl.ds((_NS - 1) * zr, zl)])

        plsc.subcore_barrier()

        wait_in(0)
        cur = ids0[pl.ds(0, NL)][0]
        accs = [jnp.zeros((NL,), jnp.float32)] * NV
        prefill(0)
        cur, cntA, accs = process_chunk(0, cur, jnp.int32(0), accs)
        scatter_stage(0, cntA)
        start_in(2, 0)

        wait_in(1)
        prefill(1)
        cur, cntB, accs = process_chunk(1, cur, jnp.int32(0), accs)
        scatter_stage(1, cntB)
        start_in(3, 1)

        def body(k, carry):
            g = 2 * k
            cur, cntA, cntB = carry[0], carry[1], carry[2]
            accs = list(carry[3:])
            wait_in(0)
            wait_stage(0, cntA)
            prefill(0)
            cur, cntA, accs = process_chunk(0, cur, jnp.int32(0), accs)
            scatter_stage(0, cntA)
            start_in(g + 2, 0)
            wait_in(1)
            wait_stage(1, cntB)
            prefill(1)
            cur, cntB, accs = process_chunk(1, cur, jnp.int32(0), accs)
            scatter_stage(1, cntB)
            start_in(g + 3, 1)
            return (cur, cntA, cntB, *accs)

        out = lax.fori_loop(1, (n_chunks - 3) // 2, body,
                            (cur, cntA, cntB, *accs))
        cur, cntA, cntB = out[0], out[1], out[2]
        accs = list(out[3:])

        wait_in(0)
        wait_stage(0, cntA)
        prefill(0)
        cur, cntA, accs = process_chunk(0, cur, jnp.int32(0), accs)
        scatter_stage(0, cntA)
        start_in(n_chunks - 1, 0)

        wait_in(1)
        wait_stage(1, cntB)
        prefill(1)
        cur, cntB, accs = process_chunk(1, cur, jnp.int32(0), accs)
        scatter_stage(1, cntB)

        wait_in(0)
        wait_stage(0, cntA)
        prefill(0)
        cur, cntA, accs = process_chunk(0, cur, jnp.int32(0), accs)
        scatter_stage(0, cntA)

        wait_stage(1, cntB)
        prefill(1)
        zero_vec = jnp.zeros((NL,), jnp.int32)
        for j in range(NV):
            plsc.store_scatter(stage_r.at[1], [zero_vec, iotas[j]], accs[j])
        plsc.store_scatter(stage_i1, [zero_vec],
                           jnp.broadcast_to(cur, (NL,)), mask=lane0)
        pltpu.async_copy(stage_r.at[1, pl.ds(0, NL)],
                         acc.at[stage_i1[pl.ds(0, NL)]], sem_sc[1], add=True)
        pltpu.make_async_copy(stage_r.at[1, pl.ds(0, NL)],
                              acc.at[dummy_vec], sem_sc[1]).wait()
        wait_stage(0, cntA)

        plsc.subcore_barrier()

        @pl.when(sid < _NS - 1)
        def _():
            pltpu.sync_copy(
                acc.at[pl.ds(sid * zr, zr)], out_hbm.at[cid, pl.ds(sid * zr, zr)])

        @pl.when(sid == _NS - 1)
        def _():
            pltpu.sync_copy(
                acc.at[pl.ds((_NS - 1) * zr, zl)],
                out_hbm.at[cid, pl.ds((_NS - 1) * zr, zl)])

    zeros = jnp.zeros((zl, D), jnp.float32)
    return seg_sum(feat_path.reshape(-1), ids, zeros)


def _tc_dense(partials, feat_center, W_type, W_enc, W_dec, W_cls, b_cls):
    N, D = feat_center.shape
    P = W_type.shape[1]
    S = W_cls.shape[1]
    BN = 2000
    assert N % BN == 0

    def body(p_ref, fc_ref, wt_ref, we_ref, wd_ref, wc_ref, bc_ref, out1_ref, out2_ref):
        wcomb = jnp.dot(we_ref[...], wd_ref[...], preferred_element_type=jnp.float32)
        seg = p_ref[0] + p_ref[1]
        inj = jnp.dot(seg, wcomb, preferred_element_type=jnp.float32) * _W1
        center = jnp.dot(fc_ref[...], wt_ref[...], preferred_element_type=jnp.float32)
        logits = jnp.dot(center + inj, wc_ref[...],
                         preferred_element_type=jnp.float32) + bc_ref[...]
        m = jnp.max(logits, axis=1, keepdims=True)
        lse = jnp.log(jnp.sum(jnp.exp(logits - m), axis=1, keepdims=True)) + m
        out1_ref[...] = logits - lse
        out2_ref[...] = inj

    return pl.pallas_call(
        body,
        grid=(N // BN,),
        in_specs=[
            pl.BlockSpec((2, BN, D), lambda i: (0, i, 0)),
            pl.BlockSpec((BN, D), lambda i: (i, 0)),
            pl.BlockSpec((D, P), lambda i: (0, 0)),
            pl.BlockSpec((D, P), lambda i: (0, 0)),
            pl.BlockSpec((P, P), lambda i: (0, 0)),
            pl.BlockSpec((P, S), lambda i: (0, 0)),
            pl.BlockSpec((1, S), lambda i: (0, 0)),
        ],
        out_specs=[
            pl.BlockSpec((BN, S), lambda i: (i, 0)),
            pl.BlockSpec((BN, P), lambda i: (i, 0)),
        ],
        out_shape=[
            jax.ShapeDtypeStruct((N, S), jnp.float32),
            jax.ShapeDtypeStruct((N, P), jnp.float32),
        ],
    )(partials, feat_center, W_type, W_enc, W_dec, W_cls, b_cls.reshape(1, S))


def kernel(feat_center, feat_path, segment_ids, W_type, W_enc, W_dec, W_cls, b_cls):
    N = feat_center.shape[0]
    ids = segment_ids.astype(jnp.int32)
    partials = _sc_segment_sum(feat_path, ids, N)
    pre_embed, inj_scaled = _tc_dense(
        partials, feat_center, W_type, W_enc, W_dec, W_cls, b_cls)
    return (pre_embed, inj_scaled)


# speedup vs baseline: 1.0135x; 1.0107x over previous
import functools
import math

import jax
import jax.numpy as jnp
from jax import lax
from jax.experimental import pallas as pl
from jax.experimental.pallas import tpu as pltpu
from jax.experimental.pallas import tpu_sc as plsc

_NC = 2
_NS = 16

_W1 = float(1.0 / (1.0 + math.exp(-0.5)))


def _sc_segment_sum(feat_path, ids, n_nodes):
    E, D = feat_path.shape
    NW = _NC * _NS
    NL = 16
    NV = D // NL
    rt = E // NW
    C = 80
    n_chunks = rt // C
    n_periods = (n_chunks - 1) // 2
    assert rt % C == 0 and C % 8 == 0 and n_chunks % 2 == 1 and n_chunks >= 9
    FB = C // NL
    zr = (n_nodes // _NS) // 8 * 8
    zl = n_nodes - zr * (_NS - 1)
    dummy = n_nodes

    mesh = plsc.VectorSubcoreMesh(core_axis_name="c", subcore_axis_name="s")

    @functools.partial(
        pl.kernel,
        mesh=mesh,
        out_type=jax.ShapeDtypeStruct((_NC, n_nodes, D), jnp.float32),
        compiler_params=pltpu.CompilerParams(needs_layout_passes=False),
        scratch_types=[
            pltpu.VMEM((C, D), jnp.float32),
            pltpu.VMEM((1, C), jnp.int32),
            pltpu.VMEM((C, D), jnp.float32),
            pltpu.VMEM((C, D), jnp.float32),
            pltpu.VMEM((1, C), jnp.int32),
            pltpu.VMEM((1, C), jnp.int32),
            pltpu.VMEM((C, D), jnp.float32),
            pltpu.VMEM((C,), jnp.int32),
            pltpu.VMEM_SHARED((n_nodes + 8, D), jnp.float32),
            pltpu.SemaphoreType.DMA,
            pltpu.SemaphoreType.DMA,
            [pltpu.SemaphoreType.DMA] * 2,
            pltpu.SemaphoreType.DMA,
        ],
    )
    def seg_sum(rows2d_hbm, ids3d_hbm, zeros_hbm,
                out_hbm, rows_s, ids_s, rows_f0, rows_f1, ids_f0, ids_f1,
                stage_r, stage_i, acc, sem_s, sem_raw, sem_f, sem_sc):
        cid = lax.axis_index("c")
        sid = lax.axis_index("s")
        wid = cid * _NS + sid
        base = wid * rt
        dummy_vec = jnp.full((NL,), dummy, jnp.int32)
        rows_fb = (rows_f0, rows_f1)
        ids_fb = (ids_f0, ids_f1)
        lane0 = lax.iota(jnp.int32, NL) == 0
        iotas = [lax.iota(jnp.int32, NL) + j * NL for j in range(NV)]

        def start_in_s(g):
            pltpu.async_copy(
                rows2d_hbm.at[pl.ds(base + g * C, C)], rows_s, sem_s)
            pltpu.async_copy(ids3d_hbm.at[wid * n_chunks + g], ids_s, sem_s)

        def wait_in_s():
            pltpu.make_async_copy(
                rows2d_hbm.at[pl.ds(0, C)], rows_s, sem_s).wait()
            pltpu.make_async_copy(ids3d_hbm.at[0], ids_s, sem_s).wait()

        def start_in_f(g, fb):
            pltpu.async_copy(
                rows2d_hbm.at[pl.ds(base + g * C, C)], rows_fb[fb], sem_f[fb])
            pltpu.async_copy(
                ids3d_hbm.at[wid * n_chunks + g], ids_fb[fb], sem_f[fb])

        def wait_in_f(fb):
            pltpu.make_async_copy(
                rows2d_hbm.at[pl.ds(0, C)], rows_fb[fb], sem_f[fb]).wait()
            pltpu.make_async_copy(ids3d_hbm.at[0], ids_fb[fb], sem_f[fb]).wait()

        def scatter_raw():
            pltpu.async_copy(rows_s, acc.at[ids_s.at[0]], sem_raw, add=True)

        def wait_raw():
            pltpu.make_async_copy(rows_s, acc.at[ids_s.at[0]], sem_raw).wait()

        def scatter_stage(cnt):
            @pl.when(cnt <= NL)
            def _():
                pltpu.async_copy(stage_r.at[pl.ds(0, NL)],
                                 acc.at[stage_i[pl.ds(0, NL)]], sem_sc, add=True)

            @pl.when(cnt > NL)
            def _():
                for j in range(FB):
                    pltpu.async_copy(stage_r.at[pl.ds(j * NL, NL)],
                                     acc.at[stage_i[pl.ds(j * NL, NL)]],
                                     sem_sc, add=True)

        def wait_stage(cntp):
            @pl.when(cntp <= NL)
            def _():
                pltpu.make_async_copy(stage_r.at[pl.ds(0, NL)],
                                      acc.at[dummy_vec], sem_sc).wait()

            @pl.when(cntp > NL)
            def _():
                for j in range(FB):
                    pltpu.make_async_copy(stage_r.at[pl.ds(j * NL, NL)],
                                          acc.at[dummy_vec], sem_sc).wait()

        def prefill():
            for j in range(FB):
                plsc.store_scatter(stage_i, [iotas[j]], dummy_vec)

        def flush(cnt, seg_id, accs):
            cnt_vec = jnp.broadcast_to(cnt, (NL,))
            for j in range(NV):
                plsc.store_scatter(stage_r, [cnt_vec, iotas[j]], accs[j])
            plsc.store_scatter(stage_i, [cnt_vec],
                               jnp.broadcast_to(seg_id, (NL,)), mask=lane0)

        def fold_chunk(fb):
            rows_b = rows_fb[fb]
            ids_b = ids_fb[fb]

            def fast(rg, carry):
                cur, cnt = carry[0], carry[1]
                accs = list(carry[2:])
                for u in range(NL):
                    r = rg * NL + u
                    accs = [accs[j] + rows_b[r, pl.ds(j * NL, NL)]
                            for j in range(NV)]
                return (cur, cnt, *accs)

            def slow(rg, carry, idvec):
                cur, cnt = carry[0], carry[1]
                accs = list(carry[2:])
                for u in range(NL):
                    r = rg * NL + u
                    idr = idvec[u]
                    same = idr == cur
                    row = [rows_b[r, pl.ds(j * NL, NL)] for j in range(NV)]
                    old = accs

                    @pl.when(jnp.logical_not(same))
                    def _(cur=cur, cnt=cnt, old=old):
                        flush(cnt, cur, old)

                    accs = [jnp.where(same, old[j] + row[j], row[j])
                            for j in range(NV)]
                    cnt = jnp.where(same, cnt, cnt + 1)
                    cur = idr
                return (cur, cnt, *accs)

            def group(rg, carry):
                idvec = ids_b[0, pl.ds(rg * NL, NL)]
                uni = jnp.all(idvec == jnp.broadcast_to(carry[0], (NL,)))
                return lax.cond(uni,
                                lambda ops: fast(rg, ops),
                                lambda ops: slow(rg, ops, idvec),
                                carry)

            cur0 = ids_b[0, pl.ds(0, NL)][0]
            accs0 = [jnp.zeros((NL,), jnp.float32)] * NV
            out = lax.fori_loop(0, C // NL, group, (cur0, jnp.int32(0), *accs0))
            cur, cnt = out[0], out[1]
            flush(cnt, cur, list(out[2:]))
            return cnt + 1

        start_in_s(0)
        start_in_f(1, 0)

        @pl.when(sid < _NS - 1)
        def _():
            pltpu.sync_copy(zeros_hbm.at[pl.ds(0, zr)], acc.at[pl.ds(sid * zr, zr)])

        @pl.when(sid == _NS - 1)
        def _():
            pltpu.sync_copy(zeros_hbm.at[pl.ds(0, zl)],
                            acc.at[pl.ds((_NS - 1) * zr, zl)])

        plsc.subcore_barrier()

        wait_in_s()
        scatter_raw()
        start_in_f(3, 1)
        wait_in_f(0)
        prefill()
        cntp = fold_chunk(0)
        scatter_stage(cntp)
        wait_raw()
        start_in_s(2)

        def period(p, fb, cntp, refill_f):
            g = 2 * p
            wait_in_s()
            scatter_raw()
            if refill_f:
                start_in_f(g + 3, 1 - fb)
            wait_in_f(fb)
            wait_stage(cntp)
            prefill()
            cntp = fold_chunk(fb)
            scatter_stage(cntp)
            wait_raw()
            start_in_s(g + 2)
            return cntp

        def body(t, cntp):
            cntp = period(2 * t + 1, 1, cntp, True)
            cntp = period(2 * t + 2, 0, cntp, True)
            return cntp

        cntp = lax.fori_loop(0, (n_periods - 2) // 2, body, cntp)
        cntp = period(n_periods - 1, 1, cntp, False)

        wait_in_s()
        scatter_raw()
        wait_raw()
        wait_stage(cntp)

        plsc.subcore_barrier()

        @pl.when(sid < _NS - 1)
        def _():
            pltpu.sync_copy(
                acc.at[pl.ds(sid * zr, zr)], out_hbm.at[cid, pl.ds(sid * zr, zr)])

        @pl.when(sid == _NS - 1)
        def _():
            pltpu.sync_copy(
                acc.at[pl.ds((_NS - 1) * zr, zl)],
                out_hbm.at[cid, pl.ds((_NS - 1) * zr, zl)])

    assert n_periods % 2 == 0
    zeros = jnp.zeros((zl, D), jnp.float32)
    return seg_sum(feat_path, ids.reshape(E // C, 1, C), zeros)


def _tc_dense(partials, feat_center, W_type, W_enc, W_dec, W_cls, b_cls):
    N, D = feat_center.shape
    P = W_type.shape[1]
    S = W_cls.shape[1]
    BN = 2000
    assert N % BN == 0

    def body(p_ref, fc_ref, wt_ref, we_ref, wd_ref, wc_ref, bc_ref, out1_ref, out2_ref):
        wcomb = jnp.dot(we_ref[...], wd_ref[...], preferred_element_type=jnp.float32)
        seg = p_ref[0] + p_ref[1]
        inj = jnp.dot(seg, wcomb, preferred_element_type=jnp.float32) * _W1
        center = jnp.dot(fc_ref[...], wt_ref[...], preferred_element_type=jnp.float32)
        logits = jnp.dot(center + inj, wc_ref[...],
                         preferred_element_type=jnp.float32) + bc_ref[...]
        m = jnp.max(logits, axis=1, keepdims=True)
        lse = jnp.log(jnp.sum(jnp.exp(logits - m), axis=1, keepdims=True)) + m
        out1_ref[...] = logits - lse
        out2_ref[...] = inj

    return pl.pallas_call(
        body,
        grid=(N // BN,),
        in_specs=[
            pl.BlockSpec((2, BN, D), lambda i: (0, i, 0)),
            pl.BlockSpec((BN, D), lambda i: (i, 0)),
            pl.BlockSpec((D, P), lambda i: (0, 0)),
            pl.BlockSpec((D, P), lambda i: (0, 0)),
            pl.BlockSpec((P, P), lambda i: (0, 0)),
            pl.BlockSpec((P, S), lambda i: (0, 0)),
            pl.BlockSpec((1, S), lambda i: (0, 0)),
        ],
        out_specs=[
            pl.BlockSpec((BN, S), lambda i: (i, 0)),
            pl.BlockSpec((BN, P), lambda i: (i, 0)),
        ],
        out_shape=[
            jax.ShapeDtypeStruct((N, S), jnp.float32),
            jax.ShapeDtypeStruct((N, P), jnp.float32),
        ],
    )(partials, feat_center, W_type, W_enc, W_dec, W_cls, b_cls.reshape(1, S))


def kernel(feat_center, feat_path, segment_ids, W_type, W_enc, W_dec, W_cls, b_cls):
    N = feat_center.shape[0]
    ids = segment_ids.astype(jnp.int32)
    partials = _sc_segment_sum(feat_path, ids, N)
    pre_embed, inj_scaled = _tc_dense(
        partials, feat_center, W_type, W_enc, W_dec, W_cls, b_cls)
    return (pre_embed, inj_scaled)


# speedup vs baseline: 1.1873x; 1.1715x over previous
import functools
import math

import jax
import jax.numpy as jnp
from jax import lax
from jax.experimental import pallas as pl
from jax.experimental.pallas import tpu as pltpu
from jax.experimental.pallas import tpu_sc as plsc

_NC = 2
_NS = 16
_NB = 4

_W1 = float(1.0 / (1.0 + math.exp(-0.5)))


def _sc_segment_sum(feat_path, ids, n_nodes):
    E, D = feat_path.shape
    NW = _NC * _NS
    rt = E // NW
    C = 80
    n_chunks = rt // C
    assert rt % C == 0 and C % 8 == 0 and n_chunks % _NB == 1 and n_chunks >= 9
    zr = (n_nodes // _NS) // 8 * 8
    zl = n_nodes - zr * (_NS - 1)

    ids3d = ids.reshape(E // C, 1, C)
    mesh = plsc.VectorSubcoreMesh(core_axis_name="c", subcore_axis_name="s")

    @functools.partial(
        pl.kernel,
        mesh=mesh,
        out_type=jax.ShapeDtypeStruct((_NC, n_nodes, D), jnp.float32),
        scratch_types=[
            pltpu.VMEM((_NB, C, D), jnp.float32),
            pltpu.VMEM((_NB, 1, C), jnp.int32),
            pltpu.VMEM_SHARED((n_nodes, D), jnp.float32),
            [pltpu.SemaphoreType.DMA] * _NB,
            [pltpu.SemaphoreType.DMA] * _NB,
        ],
    )
    def seg_sum(rows_hbm, ids_hbm, zeros_hbm, out_hbm, rows_v, ids_v, acc,
                sem_in, sem_sc):
        cid = lax.axis_index("c")
        sid = lax.axis_index("s")
        wid = cid * _NS + sid
        base = wid * rt

        def start_in(g, b):
            pltpu.async_copy(
                rows_hbm.at[pl.ds(base + g * C, C)], rows_v.at[b], sem_in[b])
            pltpu.async_copy(ids_hbm.at[wid * n_chunks + g], ids_v.at[b], sem_in[b])

        def wait_in(b):
            pltpu.make_async_copy(
                rows_hbm.at[pl.ds(0, C)], rows_v.at[b], sem_in[b]).wait()
            pltpu.make_async_copy(ids_hbm.at[0], ids_v.at[b], sem_in[b]).wait()

        def start_sc(b):
            pltpu.async_copy(
                rows_v.at[b], acc.at[ids_v.at[b, 0]], sem_sc[b], add=True)

        def wait_sc(b):
            pltpu.make_async_copy(
                rows_v.at[b], acc.at[ids_v.at[b, 0]], sem_sc[b]).wait()

        start_in(0, 0)
        start_in(1, 1)

        @pl.when(sid < _NS - 1)
        def _():
            pltpu.sync_copy(zeros_hbm.at[pl.ds(0, zr)], acc.at[pl.ds(sid * zr, zr)])

        @pl.when(sid == _NS - 1)
        def _():
            pltpu.sync_copy(zeros_hbm.at[pl.ds(0, zl)],
                            acc.at[pl.ds((_NS - 1) * zr, zl)])

        plsc.subcore_barrier()

        wait_in(0); start_sc(0); start_in(2, 2)
        wait_in(1); start_sc(1); start_in(3, 3)
        wait_in(2); start_sc(2); wait_sc(0); start_in(4, 0)
        wait_in(3); start_sc(3); wait_sc(1); start_in(5, 1)

        def body(k, carry):
            g = k * _NB
            wait_in(0); start_sc(0); wait_sc(2); start_in(g + 2, 2)
            wait_in(1); start_sc(1); wait_sc(3); start_in(g + 3, 3)
            wait_in(2); start_sc(2); wait_sc(0); start_in(g + 4, 0)
            wait_in(3); start_sc(3); wait_sc(1); start_in(g + 5, 1)
            return carry

        lax.fori_loop(1, (n_chunks - 1) // _NB - 1, body, 0)

        g = n_chunks - 5
        wait_in(0); start_sc(0); wait_sc(2); start_in(g + 2, 2)
        wait_in(1); start_sc(1); wait_sc(3); start_in(g + 3, 3)
        wait_in(2); start_sc(2); wait_sc(0); start_in(g + 4, 0)
        wait_in(3); start_sc(3); wait_sc(1)
        wait_in(0); start_sc(0); wait_sc(2)
        wait_sc(3); wait_sc(0)

        plsc.subcore_barrier()

        @pl.when(sid < _NS - 1)
        def _():
            pltpu.sync_copy(
                acc.at[pl.ds(sid * zr, zr)], out_hbm.at[cid, pl.ds(sid * zr, zr)])

        @pl.when(sid == _NS - 1)
        def _():
            pltpu.sync_copy(
                acc.at[pl.ds((_NS - 1) * zr, zl)],
                out_hbm.at[cid, pl.ds((_NS - 1) * zr, zl)])

    zeros = jnp.zeros((zl, D), jnp.float32)
    return seg_sum(feat_path, ids3d, zeros)


def _tc_dense(partials, feat_center, W_type, W_enc, W_dec, W_cls, b_cls):
    N, D = feat_center.shape
    P = W_type.shape[1]
    S = W_cls.shape[1]
    BN = 2000
    assert N % BN == 0

    def body(p_ref, fc_ref, wt_ref, we_ref, wd_ref, wc_ref, bc_ref, out1_ref, out2_ref):
        wcomb = jnp.dot(we_ref[...], wd_ref[...], preferred_element_type=jnp.float32)
        seg = p_ref[0] + p_ref[1]
        inj = jnp.dot(seg, wcomb, preferred_element_type=jnp.float32) * _W1
        center = jnp.dot(fc_ref[...], wt_ref[...], preferred_element_type=jnp.float32)
        logits = jnp.dot(center + inj, wc_ref[...],
                         preferred_element_type=jnp.float32) + bc_ref[...]
        m = jnp.max(logits, axis=1, keepdims=True)
        lse = jnp.log(jnp.sum(jnp.exp(logits - m), axis=1, keepdims=True)) + m
        out1_ref[...] = logits - lse
        out2_ref[...] = inj

    return pl.pallas_call(
        body,
        grid=(N // BN,),
        in_specs=[
            pl.BlockSpec((2, BN, D), lambda i: (0, i, 0)),
            pl.BlockSpec((BN, D), lambda i: (i, 0)),
            pl.BlockSpec((D, P), lambda i: (0, 0)),
            pl.BlockSpec((D, P), lambda i: (0, 0)),
            pl.BlockSpec((P, P), lambda i: (0, 0)),
            pl.BlockSpec((P, S), lambda i: (0, 0)),
            pl.BlockSpec((1, S), lambda i: (0, 0)),
        ],
        out_specs=[
            pl.BlockSpec((BN, S), lambda i: (i, 0)),
            pl.BlockSpec((BN, P), lambda i: (i, 0)),
        ],
        out_shape=[
            jax.ShapeDtypeStruct((N, S), jnp.float32),
            jax.ShapeDtypeStruct((N, P), jnp.float32),
        ],
    )(partials, feat_center, W_type, W_enc, W_dec, W_cls, b_cls.reshape(1, S))


def kernel(feat_center, feat_path, segment_ids, W_type, W_enc, W_dec, W_cls, b_cls):
    N = feat_center.shape[0]
    ids = segment_ids.astype(jnp.int32)
    partials = _sc_segment_sum(feat_path, ids, N)
    pre_embed, inj_scaled = _tc_dense(
        partials, feat_center, W_type, W_enc, W_dec, W_cls, b_cls)
    return (pre_embed, inj_scaled)
